# baseline (speedup 1.0000x reference)
import functools
import jax
import jax.numpy as jnp
from jax import lax
from jax.experimental import pallas as pl
from jax.experimental.pallas import tpu as pltpu

N_NODES = 10000
F_IN = 256
HEADS = 6
CH = 256
HC = HEADS * CH
ROWS = 1000


def _lin_kernel(x_ref, w_ref, a_ref, h_ref, aa_ref, cm_ref):
    h = jnp.dot(x_ref[...], w_ref[...], preferred_element_type=jnp.float32)
    h_ref[...] = h
    aa = jnp.dot(h, a_ref[...], preferred_element_type=jnp.float32)
    aa_ref[...] = aa
    cm = jnp.broadcast_to(jnp.max(aa, axis=0, keepdims=True), (8, 16))

    @pl.when(pl.program_id(0) == 0)
    def _():
        cm_ref[...] = cm

    @pl.when(pl.program_id(0) > 0)
    def _():
        cm_ref[...] = jnp.maximum(cm_ref[...], cm)


def _linear_stage(x, W, Acat):
    n = x.shape[0]
    f = x.shape[1]
    grid = n // ROWS
    return pl.pallas_call(
        _lin_kernel,
        grid=(grid,),
        in_specs=[
            pl.BlockSpec((ROWS, f), lambda i: (i, 0)),
            pl.BlockSpec((f, HC), lambda i: (0, 0)),
            pl.BlockSpec((HC, 16), lambda i: (0, 0)),
        ],
        out_specs=[
            pl.BlockSpec((ROWS, HC), lambda i: (i, 0)),
            pl.BlockSpec((ROWS, 16), lambda i: (i, 0)),
            pl.BlockSpec((8, 16), lambda i: (0, 0)),
        ],
        out_shape=[
            jax.ShapeDtypeStruct((n, HC), jnp.float32),
            jax.ShapeDtypeStruct((n, 16), jnp.float32),
            jax.ShapeDtypeStruct((8, 16), jnp.float32),
        ],
    )(x, W, Acat)


def _norm_mean(acc, den, heads_scale=1.0 / HEADS):
    parts = []
    for h in range(HEADS):
        d = den[:, h:h + 1] + 1e-16
        parts.append(acc[:, h * CH:(h + 1) * CH] / d)
    z = parts[0]
    for p in parts[1:]:
        z = z + p
    return z * heads_scale


def _mid_kernel(acc_ref, den_ref, b_ref, w_ref, a_ref, h_ref, aa_ref, cm_ref):
    z = _norm_mean(acc_ref[...], den_ref[...]) + b_ref[...]
    z = jnp.where(z > 0, z, jnp.exp(jnp.minimum(z, 0.0)) - 1.0)
    h = jnp.dot(z, w_ref[...], preferred_element_type=jnp.float32)
    h_ref[...] = h
    aa = jnp.dot(h, a_ref[...], preferred_element_type=jnp.float32)
    aa_ref[...] = aa
    cm = jnp.broadcast_to(jnp.max(aa, axis=0, keepdims=True), (8, 16))

    @pl.when(pl.program_id(0) == 0)
    def _():
        cm_ref[...] = cm

    @pl.when(pl.program_id(0) > 0)
    def _():
        cm_ref[...] = jnp.maximum(cm_ref[...], cm)


def _mid_stage(acc, den, b, W, Acat):
    n = acc.shape[0]
    grid = n // ROWS
    return pl.pallas_call(
        _mid_kernel,
        grid=(grid,),
        in_specs=[
            pl.BlockSpec((ROWS, HC), lambda i: (i, 0)),
            pl.BlockSpec((ROWS, 16), lambda i: (i, 0)),
            pl.BlockSpec((1, CH), lambda i: (0, 0)),
            pl.BlockSpec((CH, HC), lambda i: (0, 0)),
            pl.BlockSpec((HC, 16), lambda i: (0, 0)),
        ],
        out_specs=[
            pl.BlockSpec((ROWS, HC), lambda i: (i, 0)),
            pl.BlockSpec((ROWS, 16), lambda i: (i, 0)),
            pl.BlockSpec((8, 16), lambda i: (0, 0)),
        ],
        out_shape=[
            jax.ShapeDtypeStruct((n, HC), jnp.float32),
            jax.ShapeDtypeStruct((n, 16), jnp.float32),
            jax.ShapeDtypeStruct((8, 16), jnp.float32),
        ],
    )(acc, den, b, W, Acat)


def _final_kernel(acc_ref, den_ref, b_ref, o_ref):
    o_ref[...] = _norm_mean(acc_ref[...], den_ref[...]) + b_ref[...]


def _final_stage(acc, den, b):
    n = acc.shape[0]
    grid = n // ROWS
    return pl.pallas_call(
        _final_kernel,
        grid=(grid,),
        in_specs=[
            pl.BlockSpec((ROWS, HC), lambda i: (i, 0)),
            pl.BlockSpec((ROWS, 16), lambda i: (i, 0)),
            pl.BlockSpec((1, CH), lambda i: (0, 0)),
        ],
        out_specs=pl.BlockSpec((ROWS, CH), lambda i: (i, 0)),
        out_shape=jax.ShapeDtypeStruct((n, CH), jnp.float32),
    )(acc, den, b)


def _edge_phase(h, aa, cm, src, dst):
    n = h.shape[0]
    c6 = jax.nn.leaky_relu(cm[0, 0:6] + cm[0, 6:12], 0.2)
    logit = aa[src, 0:6] + aa[dst, 6:12]
    logit = jax.nn.leaky_relu(logit, 0.2) - c6[None, :]
    w = jnp.exp(logit)
    den6 = jax.ops.segment_sum(w, dst, num_segments=n)
    den = jnp.pad(den6, ((0, 0), (0, 10)), constant_values=1.0)
    msg = h[src].reshape(-1, HEADS, CH) * w[:, :, None]
    acc = jax.ops.segment_sum(msg.reshape(-1, HC), dst, num_segments=n)
    return acc, den


def _build_acat(att_src, att_dst):
    eye = jnp.eye(HEADS, dtype=jnp.float32)
    a_src = (att_src[:, None, :] * eye[:, :, None]).transpose(0, 2, 1).reshape(HC, HEADS)
    a_dst = (att_dst[:, None, :] * eye[:, :, None]).transpose(0, 2, 1).reshape(HC, HEADS)
    return jnp.concatenate([a_src, a_dst, jnp.zeros((HC, 4), jnp.float32)], axis=1)


def kernel(x, edge_index, W1, att_src1, att_dst1, b1, W2, att_src2, att_dst2, b2):
    n = x.shape[0]
    loop = jnp.arange(n, dtype=jnp.int32)
    src = jnp.concatenate([edge_index[0].astype(jnp.int32), loop])
    dst = jnp.concatenate([edge_index[1].astype(jnp.int32), loop])

    A1 = _build_acat(att_src1, att_dst1)
    A2 = _build_acat(att_src2, att_dst2)

    h1, aa1, cm1 = _linear_stage(x, W1, A1)
    acc1, den1 = _edge_phase(h1, aa1, cm1, src, dst)
    h2, aa2, cm2 = _mid_stage(acc1, den1, b1.reshape(1, CH), W2, A2)
    acc2, den2 = _edge_phase(h2, aa2, cm2, src, dst)
    return _final_stage(acc2, den2, b2.reshape(1, CH))


# speedup vs baseline: 18.2685x; 18.2685x over previous
import dataclasses
import functools
import jax
import jax.numpy as jnp
from jax import lax
from jax.experimental import pallas as pl
from jax.experimental.pallas import tpu as pltpu
from jax.experimental.pallas import tpu_sc as plsc

N_NODES = 10000
F_IN = 256
HEADS = 6
CH = 256
HC = HEADS * CH
HCD = HC + 128
ROWS = 1000

N_SC = 2
N_TILES = 16
CN = 500
CNP = 512
NPASS = 10
E_TOT = 160000 + N_NODES
E_PAD = 170240
ETILE = E_PAD // N_TILES
SEG = 2128
NSEG = ETILE // SEG
G = 16
SLAB = 4
K_WB = 8


def _lin_kernel(x_ref, w_ref, a_ref, h_ref, aa_ref, cm_ref):
    h = jnp.dot(x_ref[...], w_ref[...], preferred_element_type=jnp.float32)
    h_ref[...] = h
    aa = jnp.dot(h, a_ref[...], preferred_element_type=jnp.float32)
    aa_ref[...] = jnp.concatenate(
        [aa, jnp.zeros((aa.shape[0], 112), jnp.float32)], axis=1)
    cm = jnp.broadcast_to(jnp.max(aa, axis=0, keepdims=True), (8, 16))

    @pl.when(pl.program_id(0) == 0)
    def _():
        cm_ref[...] = cm

    @pl.when(pl.program_id(0) > 0)
    def _():
        cm_ref[...] = jnp.maximum(cm_ref[...], cm)


def _linear_stage(x, W, Acat):
    n = x.shape[0]
    f = x.shape[1]
    grid = n // ROWS
    return pl.pallas_call(
        _lin_kernel,
        grid=(grid,),
        in_specs=[
            pl.BlockSpec((ROWS, f), lambda i: (i, 0)),
            pl.BlockSpec((f, HC), lambda i: (0, 0)),
            pl.BlockSpec((HC, 16), lambda i: (0, 0)),
        ],
        out_specs=[
            pl.BlockSpec((ROWS, HC), lambda i: (i, 0)),
            pl.BlockSpec((ROWS, 128), lambda i: (i, 0)),
            pl.BlockSpec((8, 16), lambda i: (0, 0)),
        ],
        out_shape=[
            jax.ShapeDtypeStruct((n, HC), jnp.float32),
            jax.ShapeDtypeStruct((n, 128), jnp.float32),
            jax.ShapeDtypeStruct((8, 16), jnp.float32),
        ],
    )(x, W, Acat)


def _head_mean(acc):
    z = acc[:, 0:CH]
    for h in range(1, HEADS):
        z = z + acc[:, h * CH:(h + 1) * CH]
    return z * (1.0 / HEADS)


def _mid_kernel(acc_ref, b_ref, w_ref, a_ref, h_ref, aa_ref, cm_ref):
    z = _head_mean(acc_ref[...]) + b_ref[...]
    z = jnp.where(z > 0, z, jnp.exp(jnp.minimum(z, 0.0)) - 1.0)
    h = jnp.dot(z, w_ref[...], preferred_element_type=jnp.float32)
    h_ref[...] = h
    aa = jnp.dot(h, a_ref[...], preferred_element_type=jnp.float32)
    aa_ref[...] = jnp.concatenate(
        [aa, jnp.zeros((aa.shape[0], 112), jnp.float32)], axis=1)
    cm = jnp.broadcast_to(jnp.max(aa, axis=0, keepdims=True), (8, 16))

    @pl.when(pl.program_id(0) == 0)
    def _():
        cm_ref[...] = cm

    @pl.when(pl.program_id(0) > 0)
    def _():
        cm_ref[...] = jnp.maximum(cm_ref[...], cm)


def _mid_stage(acc, b, W, Acat):
    n = acc.shape[0]
    grid = n // ROWS
    return pl.pallas_call(
        _mid_kernel,
        grid=(grid,),
        in_specs=[
            pl.BlockSpec((ROWS, HC), lambda i: (i, 0)),
            pl.BlockSpec((1, CH), lambda i: (0, 0)),
            pl.BlockSpec((CH, HC), lambda i: (0, 0)),
            pl.BlockSpec((HC, 16), lambda i: (0, 0)),
        ],
        out_specs=[
            pl.BlockSpec((ROWS, HC), lambda i: (i, 0)),
            pl.BlockSpec((ROWS, 128), lambda i: (i, 0)),
            pl.BlockSpec((8, 16), lambda i: (0, 0)),
        ],
        out_shape=[
            jax.ShapeDtypeStruct((n, HC), jnp.float32),
            jax.ShapeDtypeStruct((n, 128), jnp.float32),
            jax.ShapeDtypeStruct((8, 16), jnp.float32),
        ],
    )(acc, b, W, Acat)


def _final_kernel(acc_ref, b_ref, o_ref):
    o_ref[...] = _head_mean(acc_ref[...]) + b_ref[...]


def _final_stage(acc, b):
    n = acc.shape[0]
    grid = n // ROWS
    return pl.pallas_call(
        _final_kernel,
        grid=(grid,),
        in_specs=[
            pl.BlockSpec((ROWS, HC), lambda i: (i, 0)),
            pl.BlockSpec((1, CH), lambda i: (0, 0)),
        ],
        out_specs=pl.BlockSpec((ROWS, CH), lambda i: (i, 0)),
        out_shape=jax.ShapeDtypeStruct((n, CH), jnp.float32),
    )(acc, b)


RANGE = 320
WROWS = 40
NWIN = RANGE // WROWS
N_PAD = 32 * RANGE
SEG = 2128
NSEGS = E_PAD // SEG
MCAP = 8192
FLUSH = MCAP - SEG


def _edge_body(h_hbm, aa_hbm, cb_hbm, src_hbm, dst_hbm, out_hbm,
               sseg, dseg, m_src, m_rel, dstag, asr, adr, c_v,
               hbuf, accw, denw):
    cid = lax.axis_index("c")
    sid = lax.axis_index("s")
    wid = cid * N_TILES + sid
    iot = lax.iota(jnp.int32, 16)
    zv = jnp.zeros((16,), jnp.float32)

    pltpu.sync_copy(cb_hbm, c_v)

    def _process(cnt, lo):
        m_src[pl.ds(cnt, 16)] = jnp.zeros((16,), jnp.int32)
        m_rel[pl.ds(cnt, 16)] = jnp.zeros((16,), jnp.int32)
        ngroups = (cnt + 15) // 16

        def _group(g, carry):
            gb = g * 16
            relv = m_rel[pl.ds(gb, 16)]
            dstag[...] = relv + lo
            pltpu.sync_copy(aa_hbm.at[m_src.at[pl.ds(gb, 16)]], asr)
            pltpu.sync_copy(aa_hbm.at[dstag], adr)
            pltpu.sync_copy(h_hbm.at[m_src.at[pl.ds(gb, 16)]], hbuf)

            ws = []
            for h in range(HEADS):
                s = plsc.load_gather(asr, [iot, jnp.full((16,), 1 + h, jnp.int32)])
                d = plsc.load_gather(adr, [iot, jnp.full((16,), 7 + h, jnp.int32)])
                lg = s + d
                lg = jnp.where(lg > 0, lg, 0.2 * lg)
                cv = plsc.load_gather(c_v, [jnp.full((16,), 8 + h, jnp.int32)])
                ws.append(jnp.exp(lg - cv))

            ne = jnp.minimum(cnt - gb, 16)

            def _edge(e, carry2):
                rel = jnp.sum(jnp.where(iot == e, relv, 0))
                sv = [jnp.sum(jnp.where(iot == e, ws[h], 0.0)) for h in range(HEADS)]
                wcol = jnp.zeros((16,), jnp.float32)
                for h in range(HEADS):
                    wcol = jnp.where(iot == h, sv[h], wcol)
                denw[rel, :] = denw[rel, :] + wcol
                for h in range(HEADS):
                    wv = jnp.full((16,), sv[h])
                    for j in range(CH // 16):
                        c0 = h * CH + j * 16
                        accw[rel, pl.ds(c0, 16)] = (
                            accw[rel, pl.ds(c0, 16)] + hbuf[e, pl.ds(c0, 16)] * wv)
                return carry2

            lax.fori_loop(0, ne, _edge, jnp.int32(0))
            return carry

        lax.fori_loop(0, ngroups, _group, jnp.int32(0))

    @pl.loop(0, NWIN)
    def _window(w):
        lo = wid * RANGE + w * WROWS

        @pl.loop(0, WROWS)
        def _zero(r):
            denw[r, :] = zv
            for j in range(HC // 16):
                accw[r, pl.ds(j * 16, 16)] = zv

        def _seg(s, cnt):
            pltpu.sync_copy(src_hbm.at[pl.ds(s * SEG, SEG)], sseg)
            pltpu.sync_copy(dst_hbm.at[pl.ds(s * SEG, SEG)], dseg)

            def _compact(i, c):
                d = dseg[pl.ds(i * 16, 16)]
                sv = sseg[pl.ds(i * 16, 16)]
                m = (d >= lo) & (d < lo + WROWS)
                plsc.store_compressed(m_rel.at[pl.ds(c, 16)], d - lo, mask=m)
                plsc.store_compressed(m_src.at[pl.ds(c, 16)], sv, mask=m)
                return c + jnp.sum(m.astype(jnp.int32))

            cnt = lax.fori_loop(0, SEG // 16, _compact, cnt)

            @pl.when(cnt >= FLUSH)
            def _():
                _process(cnt, lo)

            return jnp.where(cnt >= FLUSH, jnp.int32(0), cnt)

        cnt = lax.fori_loop(0, NSEGS, _seg, jnp.int32(0))
        _process(cnt, lo)

        @pl.loop(0, WROWS)
        def _div(r):
            rv16 = jnp.full((16,), r, jnp.int32)
            for h in range(HEADS):
                dv = plsc.load_gather(denw, [rv16, jnp.full((16,), h, jnp.int32)])
                rv = 1.0 / (dv + 1e-16)
                for j in range(CH // 16):
                    c0 = h * CH + j * 16
                    accw[r, pl.ds(c0, 16)] = accw[r, pl.ds(c0, 16)] * rv

        pltpu.sync_copy(accw, out_hbm.at[pl.ds(lo, WROWS)])


def _edge_phase_sc(h, aa, cbound, srcp, dstp):
    aa = jnp.pad(aa, ((0, N_PAD - N_NODES), (0, 0)))
    mesh = plsc.VectorSubcoreMesh(core_axis_name="c", subcore_axis_name="s")
    cp = pltpu.CompilerParams()
    if "needs_layout_passes" in pltpu.CompilerParams.__dataclass_fields__:
        cp = dataclasses.replace(cp, needs_layout_passes=False)
    kfn = pl.kernel(
        _edge_body,
        out_type=jax.ShapeDtypeStruct((N_PAD, HC), jnp.float32),
        mesh=mesh,
        compiler_params=cp,
        scratch_types=[
            pltpu.VMEM((SEG,), jnp.int32),
            pltpu.VMEM((SEG,), jnp.int32),
            pltpu.VMEM((MCAP + 16,), jnp.int32),
            pltpu.VMEM((MCAP + 16,), jnp.int32),
            pltpu.VMEM((16,), jnp.int32),
            pltpu.VMEM((16, 128), jnp.float32),
            pltpu.VMEM((16, 128), jnp.float32),
            pltpu.VMEM((16,), jnp.float32),
            pltpu.VMEM((16, HC), jnp.float32),
            pltpu.VMEM((WROWS, HC), jnp.float32),
            pltpu.VMEM((WROWS, 16), jnp.float32),
        ],
    )
    return kfn(h, aa, cbound, srcp, dstp)[0:N_NODES]


def _build_acat(att_src, att_dst):
    eye = jnp.eye(HEADS, dtype=jnp.float32)
    a_src = (att_src[:, None, :] * eye[:, :, None]).transpose(0, 2, 1).reshape(HC, HEADS)
    a_dst = (att_dst[:, None, :] * eye[:, :, None]).transpose(0, 2, 1).reshape(HC, HEADS)
    return jnp.concatenate([jnp.zeros((HC, 1), jnp.float32), a_src, a_dst,
                            jnp.zeros((HC, 3), jnp.float32)], axis=1)


def _cbound(cm):
    c6 = cm[0, 1:7] + cm[0, 7:13]
    c6 = jnp.where(c6 > 0, c6, 0.2 * c6)
    return jnp.pad(c6, (8, 2))


def kernel(x, edge_index, W1, att_src1, att_dst1, b1, W2, att_src2, att_dst2, b2):
    n = x.shape[0]
    loop = jnp.arange(n, dtype=jnp.int32)
    pad = E_PAD - E_TOT
    srcp = jnp.concatenate([edge_index[0].astype(jnp.int32), loop,
                            jnp.zeros((pad,), jnp.int32)])
    dstp = jnp.concatenate([edge_index[1].astype(jnp.int32), loop,
                            jnp.full((pad,), 2 * N_NODES, jnp.int32)])

    A1 = _build_acat(att_src1, att_dst1)
    A2 = _build_acat(att_src2, att_dst2)

    h1, aa1, cm1 = _linear_stage(x, W1, A1)
    acc1 = _edge_phase_sc(h1, aa1, _cbound(cm1), srcp, dstp)
    h2, aa2, cm2 = _mid_stage(acc1, b1.reshape(1, CH), W2, A2)
    acc2 = _edge_phase_sc(h2, aa2, _cbound(cm2), srcp, dstp)
    return _final_stage(acc2, b2.reshape(1, CH))


# speedup vs baseline: 20.6884x; 1.1325x over previous
import dataclasses
import functools
import jax
import jax.numpy as jnp
from jax import lax
from jax.experimental import pallas as pl
from jax.experimental.pallas import tpu as pltpu
from jax.experimental.pallas import tpu_sc as plsc

N_NODES = 10000
F_IN = 256
HEADS = 6
CH = 256
HC = HEADS * CH
HCD = HC + 128
ROWS = 1000

N_SC = 2
N_TILES = 16
CN = 500
CNP = 512
NPASS = 10
E_TOT = 160000 + N_NODES
E_PAD = 170240
ETILE = E_PAD // N_TILES
SEG = 2128
NSEG = ETILE // SEG
G = 16
SLAB = 4
K_WB = 8


def _lin_kernel(x_ref, w_ref, a_ref, h_ref, aa_ref, cm_ref):
    h = jnp.dot(x_ref[...], w_ref[...], preferred_element_type=jnp.float32)
    h_ref[...] = h
    aa = jnp.dot(h, a_ref[...], preferred_element_type=jnp.float32)
    aa_ref[...] = jnp.concatenate(
        [aa, jnp.zeros((aa.shape[0], 112), jnp.float32)], axis=1)
    cm = jnp.broadcast_to(jnp.max(aa, axis=0, keepdims=True), (8, 16))

    @pl.when(pl.program_id(0) == 0)
    def _():
        cm_ref[...] = cm

    @pl.when(pl.program_id(0) > 0)
    def _():
        cm_ref[...] = jnp.maximum(cm_ref[...], cm)


def _linear_stage(x, W, Acat):
    n = x.shape[0]
    f = x.shape[1]
    grid = n // ROWS
    return pl.pallas_call(
        _lin_kernel,
        grid=(grid,),
        in_specs=[
            pl.BlockSpec((ROWS, f), lambda i: (i, 0)),
            pl.BlockSpec((f, HC), lambda i: (0, 0)),
            pl.BlockSpec((HC, 16), lambda i: (0, 0)),
        ],
        out_specs=[
            pl.BlockSpec((ROWS, HC), lambda i: (i, 0)),
            pl.BlockSpec((ROWS, 128), lambda i: (i, 0)),
            pl.BlockSpec((8, 16), lambda i: (0, 0)),
        ],
        out_shape=[
            jax.ShapeDtypeStruct((n, HC), jnp.float32),
            jax.ShapeDtypeStruct((n, 128), jnp.float32),
            jax.ShapeDtypeStruct((8, 16), jnp.float32),
        ],
    )(x, W, Acat)


def _head_mean(acc):
    z = acc[:, 0:CH]
    for h in range(1, HEADS):
        z = z + acc[:, h * CH:(h + 1) * CH]
    return z * (1.0 / HEADS)


def _mid_kernel(acc_ref, b_ref, w_ref, a_ref, h_ref, aa_ref, cm_ref):
    z = _head_mean(acc_ref[...]) + b_ref[...]
    z = jnp.where(z > 0, z, jnp.exp(jnp.minimum(z, 0.0)) - 1.0)
    h = jnp.dot(z, w_ref[...], preferred_element_type=jnp.float32)
    h_ref[...] = h
    aa = jnp.dot(h, a_ref[...], preferred_element_type=jnp.float32)
    aa_ref[...] = jnp.concatenate(
        [aa, jnp.zeros((aa.shape[0], 112), jnp.float32)], axis=1)
    cm = jnp.broadcast_to(jnp.max(aa, axis=0, keepdims=True), (8, 16))

    @pl.when(pl.program_id(0) == 0)
    def _():
        cm_ref[...] = cm

    @pl.when(pl.program_id(0) > 0)
    def _():
        cm_ref[...] = jnp.maximum(cm_ref[...], cm)


def _mid_stage(acc, b, W, Acat):
    n = acc.shape[0]
    grid = n // ROWS
    return pl.pallas_call(
        _mid_kernel,
        grid=(grid,),
        in_specs=[
            pl.BlockSpec((ROWS, HC), lambda i: (i, 0)),
            pl.BlockSpec((1, CH), lambda i: (0, 0)),
            pl.BlockSpec((CH, HC), lambda i: (0, 0)),
            pl.BlockSpec((HC, 16), lambda i: (0, 0)),
        ],
        out_specs=[
            pl.BlockSpec((ROWS, HC), lambda i: (i, 0)),
            pl.BlockSpec((ROWS, 128), lambda i: (i, 0)),
            pl.BlockSpec((8, 16), lambda i: (0, 0)),
        ],
        out_shape=[
            jax.ShapeDtypeStruct((n, HC), jnp.float32),
            jax.ShapeDtypeStruct((n, 128), jnp.float32),
            jax.ShapeDtypeStruct((8, 16), jnp.float32),
        ],
    )(acc, b, W, Acat)


def _final_kernel(acc_ref, b_ref, o_ref):
    o_ref[...] = _head_mean(acc_ref[...]) + b_ref[...]


def _final_stage(acc, b):
    n = acc.shape[0]
    grid = n // ROWS
    return pl.pallas_call(
        _final_kernel,
        grid=(grid,),
        in_specs=[
            pl.BlockSpec((ROWS, HC), lambda i: (i, 0)),
            pl.BlockSpec((1, CH), lambda i: (0, 0)),
        ],
        out_specs=pl.BlockSpec((ROWS, CH), lambda i: (i, 0)),
        out_shape=jax.ShapeDtypeStruct((n, CH), jnp.float32),
    )(acc, b)


RANGE = 320
WROWS = 40
NWIN = RANGE // WROWS
N_PAD = 32 * RANGE
SEG = 2128
NSEGS = E_PAD // SEG
MCAP = 8192
FLUSH = MCAP - SEG


def _edge_body(h_hbm, aa_hbm, cb_hbm, src_hbm, dst_hbm, out_hbm,
               sseg, dseg, sseg2, dseg2, m_src, m_rel, dstag, asr, adr, wbuf,
               c_v, hbuf, accw, denw, sem_a, sem_b):
    cid = lax.axis_index("c")
    sid = lax.axis_index("s")
    wid = cid * N_TILES + sid
    iot = lax.iota(jnp.int32, 16)
    zv = jnp.zeros((16,), jnp.float32)

    pltpu.sync_copy(cb_hbm, c_v)

    def _process(cnt, lo):
        m_src[pl.ds(cnt, 16)] = jnp.zeros((16,), jnp.int32)
        m_rel[pl.ds(cnt, 16)] = jnp.zeros((16,), jnp.int32)
        ngroups = (cnt + 15) // 16

        def _group(g, carry):
            gb = g * 16
            relv = m_rel[pl.ds(gb, 16)]
            dstag[...] = relv + lo
            pltpu.sync_copy(aa_hbm.at[m_src.at[pl.ds(gb, 16)]], asr)
            pltpu.sync_copy(aa_hbm.at[dstag], adr)
            pltpu.sync_copy(h_hbm.at[m_src.at[pl.ds(gb, 16)]], hbuf)

            for h in range(HEADS):
                s = plsc.load_gather(asr, [iot, jnp.full((16,), 1 + h, jnp.int32)])
                d = plsc.load_gather(adr, [iot, jnp.full((16,), 7 + h, jnp.int32)])
                lg = s + d
                lg = jnp.where(lg > 0, lg, 0.2 * lg)
                cv = plsc.load_gather(c_v, [jnp.full((16,), 8 + h, jnp.int32)])
                wbuf[1 + h, pl.ds(8, 16)] = jnp.exp(lg - cv)

            ne = jnp.minimum(cnt - gb, 16)

            def _edge(e, carry2):
                rel = jnp.sum(jnp.where(iot == e, relv, 0))
                ev = jnp.full((16,), 8, jnp.int32) + e
                wcol = plsc.load_gather(wbuf, [1 + jnp.minimum(iot, 5), ev])
                wcol = jnp.where(iot < 6, wcol, 0.0)
                denw[rel, :] = denw[rel, :] + wcol
                for h in range(HEADS):
                    wv = plsc.load_gather(wbuf, [jnp.full((16,), 1 + h, jnp.int32), ev])
                    for j in range(CH // 16):
                        c0 = h * CH + j * 16
                        accw[rel, pl.ds(c0, 16)] = (
                            accw[rel, pl.ds(c0, 16)] + hbuf[e, pl.ds(c0, 16)] * wv)
                return carry2

            lax.fori_loop(0, ne, _edge, jnp.int32(0))
            return carry

        lax.fori_loop(0, ngroups, _group, jnp.int32(0))

    @pl.loop(0, NWIN)
    def _window(w):
        lo = wid * RANGE + w * WROWS

        @pl.loop(0, WROWS)
        def _zero(r):
            denw[r, :] = zv
            for j in range(HC // 16):
                accw[r, pl.ds(j * 16, 16)] = zv

        def _compact_buf(dref, sref, cnt):
            def _compact(i, c):
                d = dref[pl.ds(i * 16, 16)]
                sv = sref[pl.ds(i * 16, 16)]
                m = (d >= lo) & (d < lo + WROWS)
                plsc.store_compressed(m_rel.at[pl.ds(c, 16)], d - lo, mask=m)
                plsc.store_compressed(m_src.at[pl.ds(c, 16)], sv, mask=m)
                return c + jnp.sum(m.astype(jnp.int32))

            cnt = lax.fori_loop(0, SEG // 16, _compact, cnt)

            @pl.when(cnt >= FLUSH)
            def _():
                _process(cnt, lo)

            return jnp.where(cnt >= FLUSH, jnp.int32(0), cnt)

        def _issue(s, sbuf, dbuf, sem):
            pltpu.make_async_copy(src_hbm.at[pl.ds(s * SEG, SEG)], sbuf, sem).start()
            pltpu.make_async_copy(dst_hbm.at[pl.ds(s * SEG, SEG)], dbuf, sem).start()

        def _drain(s, sbuf, dbuf, sem):
            pltpu.make_async_copy(src_hbm.at[pl.ds(s * SEG, SEG)], sbuf, sem).wait()
            pltpu.make_async_copy(dst_hbm.at[pl.ds(s * SEG, SEG)], dbuf, sem).wait()

        _issue(0, sseg, dseg, sem_a)

        def _seg2(i, cnt):
            s0 = i * 2

            @pl.when(s0 + 1 < NSEGS)
            def _():
                _issue(s0 + 1, sseg2, dseg2, sem_b)

            _drain(s0, sseg, dseg, sem_a)
            cnt = _compact_buf(dseg, sseg, cnt)

            @pl.when(s0 + 2 < NSEGS)
            def _():
                _issue(s0 + 2, sseg, dseg, sem_a)

            _drain(s0 + 1, sseg2, dseg2, sem_b)
            cnt = _compact_buf(dseg2, sseg2, cnt)
            return cnt

        cnt = lax.fori_loop(0, NSEGS // 2, _seg2, jnp.int32(0))
        _process(cnt, lo)

        @pl.loop(0, WROWS)
        def _div(r):
            rv16 = jnp.full((16,), r, jnp.int32)
            for h in range(HEADS):
                dv = plsc.load_gather(denw, [rv16, jnp.full((16,), h, jnp.int32)])
                rv = 1.0 / (dv + 1e-16)
                for j in range(CH // 16):
                    c0 = h * CH + j * 16
                    accw[r, pl.ds(c0, 16)] = accw[r, pl.ds(c0, 16)] * rv

        pltpu.sync_copy(accw, out_hbm.at[pl.ds(lo, WROWS)])


def _edge_phase_sc(h, aa, cbound, srcp, dstp):
    aa = jnp.pad(aa, ((0, N_PAD - N_NODES), (0, 0)))
    mesh = plsc.VectorSubcoreMesh(core_axis_name="c", subcore_axis_name="s")
    cp = pltpu.CompilerParams()
    if "needs_layout_passes" in pltpu.CompilerParams.__dataclass_fields__:
        cp = dataclasses.replace(cp, needs_layout_passes=False)
    kfn = pl.kernel(
        _edge_body,
        out_type=jax.ShapeDtypeStruct((N_PAD, HC), jnp.float32),
        mesh=mesh,
        compiler_params=cp,
        scratch_types=[
            pltpu.VMEM((SEG,), jnp.int32),
            pltpu.VMEM((SEG,), jnp.int32),
            pltpu.VMEM((SEG,), jnp.int32),
            pltpu.VMEM((SEG,), jnp.int32),
            pltpu.VMEM((MCAP + 16,), jnp.int32),
            pltpu.VMEM((MCAP + 16,), jnp.int32),
            pltpu.VMEM((16,), jnp.int32),
            pltpu.VMEM((16, 128), jnp.float32),
            pltpu.VMEM((16, 128), jnp.float32),
            pltpu.VMEM((8, 32), jnp.float32),
            pltpu.VMEM((16,), jnp.float32),
            pltpu.VMEM((16, HC), jnp.float32),
            pltpu.VMEM((WROWS, HC), jnp.float32),
            pltpu.VMEM((WROWS, 16), jnp.float32),
            pltpu.SemaphoreType.DMA,
            pltpu.SemaphoreType.DMA,
        ],
    )
    return kfn(h, aa, cbound, srcp, dstp)[0:N_NODES]


def _build_acat(att_src, att_dst):
    eye = jnp.eye(HEADS, dtype=jnp.float32)
    a_src = (att_src[:, None, :] * eye[:, :, None]).transpose(0, 2, 1).reshape(HC, HEADS)
    a_dst = (att_dst[:, None, :] * eye[:, :, None]).transpose(0, 2, 1).reshape(HC, HEADS)
    return jnp.concatenate([jnp.zeros((HC, 1), jnp.float32), a_src, a_dst,
                            jnp.zeros((HC, 3), jnp.float32)], axis=1)


def _cbound(cm):
    c6 = cm[0, 1:7] + cm[0, 7:13]
    c6 = jnp.where(c6 > 0, c6, 0.2 * c6)
    return jnp.pad(c6, (8, 2))


def kernel(x, edge_index, W1, att_src1, att_dst1, b1, W2, att_src2, att_dst2, b2):
    n = x.shape[0]
    loop = jnp.arange(n, dtype=jnp.int32)
    pad = E_PAD - E_TOT
    srcp = jnp.concatenate([edge_index[0].astype(jnp.int32), loop,
                            jnp.zeros((pad,), jnp.int32)])
    dstp = jnp.concatenate([edge_index[1].astype(jnp.int32), loop,
                            jnp.full((pad,), 2 * N_NODES, jnp.int32)])

    A1 = _build_acat(att_src1, att_dst1)
    A2 = _build_acat(att_src2, att_dst2)

    h1, aa1, cm1 = _linear_stage(x, W1, A1)
    acc1 = _edge_phase_sc(h1, aa1, _cbound(cm1), srcp, dstp)
    h2, aa2, cm2 = _mid_stage(acc1, b1.reshape(1, CH), W2, A2)
    acc2 = _edge_phase_sc(h2, aa2, _cbound(cm2), srcp, dstp)
    return _final_stage(acc2, b2.reshape(1, CH))


# speedup vs baseline: 24.9108x; 1.2041x over previous
import dataclasses
import functools
import jax
import jax.numpy as jnp
from jax import lax
from jax.experimental import pallas as pl
from jax.experimental.pallas import tpu as pltpu
from jax.experimental.pallas import tpu_sc as plsc

N_NODES = 10000
F_IN = 256
HEADS = 6
CH = 256
HC = HEADS * CH
HCD = HC + 128
ROWS = 1000

N_SC = 2
N_TILES = 16
CN = 500
CNP = 512
NPASS = 10
E_TOT = 160000 + N_NODES
E_PAD = 170240
ETILE = E_PAD // N_TILES
SEG = 2128
NSEG = ETILE // SEG
G = 16
SLAB = 4
K_WB = 8


def _lin_kernel(x_ref, w_ref, a_ref, h_ref, aa_ref, cm_ref):
    h = jnp.dot(x_ref[...], w_ref[...], preferred_element_type=jnp.float32)
    h_ref[...] = h
    aa = jnp.dot(h, a_ref[...], preferred_element_type=jnp.float32)
    aa_ref[...] = jnp.concatenate(
        [aa, jnp.zeros((aa.shape[0], 112), jnp.float32)], axis=1)
    cm = jnp.broadcast_to(jnp.max(aa, axis=0, keepdims=True), (8, 16))

    @pl.when(pl.program_id(0) == 0)
    def _():
        cm_ref[...] = cm

    @pl.when(pl.program_id(0) > 0)
    def _():
        cm_ref[...] = jnp.maximum(cm_ref[...], cm)


def _linear_stage(x, W, Acat):
    n = x.shape[0]
    f = x.shape[1]
    grid = n // ROWS
    return pl.pallas_call(
        _lin_kernel,
        grid=(grid,),
        in_specs=[
            pl.BlockSpec((ROWS, f), lambda i: (i, 0)),
            pl.BlockSpec((f, HC), lambda i: (0, 0)),
            pl.BlockSpec((HC, 16), lambda i: (0, 0)),
        ],
        out_specs=[
            pl.BlockSpec((ROWS, HC), lambda i: (i, 0)),
            pl.BlockSpec((ROWS, 128), lambda i: (i, 0)),
            pl.BlockSpec((8, 16), lambda i: (0, 0)),
        ],
        out_shape=[
            jax.ShapeDtypeStruct((n, HC), jnp.float32),
            jax.ShapeDtypeStruct((n, 128), jnp.float32),
            jax.ShapeDtypeStruct((8, 16), jnp.float32),
        ],
    )(x, W, Acat)


def _head_mean(acc):
    z = acc[:, 0:CH]
    for h in range(1, HEADS):
        z = z + acc[:, h * CH:(h + 1) * CH]
    return z * (1.0 / HEADS)


def _mid_kernel(acc_ref, b_ref, w_ref, a_ref, h_ref, aa_ref, cm_ref):
    z = _head_mean(acc_ref[...]) + b_ref[...]
    z = jnp.where(z > 0, z, jnp.exp(jnp.minimum(z, 0.0)) - 1.0)
    h = jnp.dot(z, w_ref[...], preferred_element_type=jnp.float32)
    h_ref[...] = h
    aa = jnp.dot(h, a_ref[...], preferred_element_type=jnp.float32)
    aa_ref[...] = jnp.concatenate(
        [aa, jnp.zeros((aa.shape[0], 112), jnp.float32)], axis=1)
    cm = jnp.broadcast_to(jnp.max(aa, axis=0, keepdims=True), (8, 16))

    @pl.when(pl.program_id(0) == 0)
    def _():
        cm_ref[...] = cm

    @pl.when(pl.program_id(0) > 0)
    def _():
        cm_ref[...] = jnp.maximum(cm_ref[...], cm)


def _mid_stage(acc, b, W, Acat):
    n = acc.shape[0]
    grid = n // ROWS
    return pl.pallas_call(
        _mid_kernel,
        grid=(grid,),
        in_specs=[
            pl.BlockSpec((ROWS, HC), lambda i: (i, 0)),
            pl.BlockSpec((1, CH), lambda i: (0, 0)),
            pl.BlockSpec((CH, HC), lambda i: (0, 0)),
            pl.BlockSpec((HC, 16), lambda i: (0, 0)),
        ],
        out_specs=[
            pl.BlockSpec((ROWS, HC), lambda i: (i, 0)),
            pl.BlockSpec((ROWS, 128), lambda i: (i, 0)),
            pl.BlockSpec((8, 16), lambda i: (0, 0)),
        ],
        out_shape=[
            jax.ShapeDtypeStruct((n, HC), jnp.float32),
            jax.ShapeDtypeStruct((n, 128), jnp.float32),
            jax.ShapeDtypeStruct((8, 16), jnp.float32),
        ],
    )(acc, b, W, Acat)


def _final_kernel(acc_ref, b_ref, o_ref):
    o_ref[...] = _head_mean(acc_ref[...]) + b_ref[...]


def _final_stage(acc, b):
    n = acc.shape[0]
    grid = n // ROWS
    return pl.pallas_call(
        _final_kernel,
        grid=(grid,),
        in_specs=[
            pl.BlockSpec((ROWS, HC), lambda i: (i, 0)),
            pl.BlockSpec((1, CH), lambda i: (0, 0)),
        ],
        out_specs=pl.BlockSpec((ROWS, CH), lambda i: (i, 0)),
        out_shape=jax.ShapeDtypeStruct((n, CH), jnp.float32),
    )(acc, b)


RANGE = 320
WROWS = 40
NWIN = RANGE // WROWS
N_PAD = 32 * RANGE
SEG = 2128
NSEGS = E_PAD // SEG
MCAP = 8192
FLUSH = MCAP - SEG


def _edge_body(h_hbm, aa_hbm, cb_hbm, src_hbm, dst_hbm, out_hbm,
               sseg, dseg, sseg2, dseg2, m_src, m_rel, dstag, dstag2,
               asr, adr, asr2, adr2, wbuf, c_v, hbuf, hbuf2, accw, denw,
               sem_a, sem_b, sem_c, sem_d):
    cid = lax.axis_index("c")
    sid = lax.axis_index("s")
    wid = cid * N_TILES + sid
    iot = lax.iota(jnp.int32, 16)
    zv = jnp.zeros((16,), jnp.float32)

    pltpu.sync_copy(cb_hbm, c_v)

    def _process(cnt, lo):
        m_src[pl.ds(cnt, 16)] = jnp.zeros((16,), jnp.int32)
        m_rel[pl.ds(cnt, 16)] = jnp.zeros((16,), jnp.int32)
        ngroups = (cnt + 7) // 8

        def _issue_g(gb, hb, ar, ad, dstg, sem):
            dstg[...] = m_rel[pl.ds(gb, 16)] + lo
            pltpu.make_async_copy(aa_hbm.at[m_src.at[pl.ds(gb, 8)]], ar, sem).start()
            pltpu.make_async_copy(aa_hbm.at[dstg], ad, sem).start()
            pltpu.make_async_copy(h_hbm.at[m_src.at[pl.ds(gb, 8)]], hb, sem).start()

        def _proc_g(gb, hb, ar, ad, dstg, sem):
            pltpu.make_async_copy(aa_hbm.at[m_src.at[pl.ds(gb, 8)]], ar, sem).wait()
            pltpu.make_async_copy(aa_hbm.at[dstg], ad, sem).wait()
            pltpu.make_async_copy(h_hbm.at[m_src.at[pl.ds(gb, 8)]], hb, sem).wait()
            relv = m_rel[pl.ds(gb, 16)]
            er = jnp.bitwise_and(iot, 7)

            for h in range(HEADS):
                s = plsc.load_gather(ar, [er, jnp.full((16,), 1 + h, jnp.int32)])
                d = plsc.load_gather(ad, [iot, jnp.full((16,), 7 + h, jnp.int32)])
                lg = s + d
                lg = jnp.where(lg > 0, lg, 0.2 * lg)
                cv = plsc.load_gather(c_v, [jnp.full((16,), 8 + h, jnp.int32)])
                wbuf[1 + h, pl.ds(8, 16)] = jnp.exp(lg - cv)

            ne = jnp.minimum(cnt - gb, 8)

            def _edge(e, carry2):
                rel = jnp.sum(jnp.where(iot == e, relv, 0))
                ev = jnp.full((16,), 8, jnp.int32) + e
                wcol = plsc.load_gather(wbuf, [1 + jnp.minimum(iot, 5), ev])
                wcol = jnp.where(iot < 6, wcol, 0.0)
                denw[rel, :] = denw[rel, :] + wcol
                for h in range(HEADS):
                    wv = plsc.load_gather(wbuf, [jnp.full((16,), 1 + h, jnp.int32), ev])
                    for j in range(CH // 16):
                        c0 = h * CH + j * 16
                        accw[rel, pl.ds(c0, 16)] = (
                            accw[rel, pl.ds(c0, 16)] + hb[e, pl.ds(c0, 16)] * wv)
                return carry2

            lax.fori_loop(0, ne, _edge, jnp.int32(0))

        @pl.when(ngroups > 0)
        def _():
            _issue_g(0, hbuf, asr, adr, dstag, sem_c)

        def _g2(i, carry):
            gb0 = i * 16

            @pl.when(2 * i + 1 < ngroups)
            def _():
                _issue_g(gb0 + 8, hbuf2, asr2, adr2, dstag2, sem_d)

            _proc_g(gb0, hbuf, asr, adr, dstag, sem_c)

            @pl.when(2 * i + 2 < ngroups)
            def _():
                _issue_g(gb0 + 16, hbuf, asr, adr, dstag, sem_c)

            @pl.when(2 * i + 1 < ngroups)
            def _():
                _proc_g(gb0 + 8, hbuf2, asr2, adr2, dstag2, sem_d)

            return carry

        lax.fori_loop(0, (ngroups + 1) // 2, _g2, jnp.int32(0))

    @pl.loop(0, NWIN)
    def _window(w):
        lo = wid * RANGE + w * WROWS

        @pl.loop(0, WROWS)
        def _zero(r):
            denw[r, :] = zv
            for j in range(HC // 16):
                accw[r, pl.ds(j * 16, 16)] = zv

        def _compact_buf(dref, sref, cnt):
            def _compact(i, c):
                d = dref[pl.ds(i * 16, 16)]
                sv = sref[pl.ds(i * 16, 16)]
                m = (d >= lo) & (d < lo + WROWS)
                plsc.store_compressed(m_rel.at[pl.ds(c, 16)], d - lo, mask=m)
                plsc.store_compressed(m_src.at[pl.ds(c, 16)], sv, mask=m)
                return c + jnp.sum(m.astype(jnp.int32))

            cnt = lax.fori_loop(0, SEG // 16, _compact, cnt)

            @pl.when(cnt >= FLUSH)
            def _():
                _process(cnt, lo)

            return jnp.where(cnt >= FLUSH, jnp.int32(0), cnt)

        def _issue(s, sbuf, dbuf, sem):
            pltpu.make_async_copy(src_hbm.at[pl.ds(s * SEG, SEG)], sbuf, sem).start()
            pltpu.make_async_copy(dst_hbm.at[pl.ds(s * SEG, SEG)], dbuf, sem).start()

        def _drain(s, sbuf, dbuf, sem):
            pltpu.make_async_copy(src_hbm.at[pl.ds(s * SEG, SEG)], sbuf, sem).wait()
            pltpu.make_async_copy(dst_hbm.at[pl.ds(s * SEG, SEG)], dbuf, sem).wait()

        _issue(0, sseg, dseg, sem_a)

        def _seg2(i, cnt):
            s0 = i * 2

            @pl.when(s0 + 1 < NSEGS)
            def _():
                _issue(s0 + 1, sseg2, dseg2, sem_b)

            _drain(s0, sseg, dseg, sem_a)
            cnt = _compact_buf(dseg, sseg, cnt)

            @pl.when(s0 + 2 < NSEGS)
            def _():
                _issue(s0 + 2, sseg, dseg, sem_a)

            _drain(s0 + 1, sseg2, dseg2, sem_b)
            cnt = _compact_buf(dseg2, sseg2, cnt)
            return cnt

        cnt = lax.fori_loop(0, NSEGS // 2, _seg2, jnp.int32(0))
        _process(cnt, lo)

        @pl.loop(0, WROWS)
        def _div(r):
            rv16 = jnp.full((16,), r, jnp.int32)
            for h in range(HEADS):
                dv = plsc.load_gather(denw, [rv16, jnp.full((16,), h, jnp.int32)])
                rv = 1.0 / (dv + 1e-16)
                for j in range(CH // 16):
                    c0 = h * CH + j * 16
                    accw[r, pl.ds(c0, 16)] = accw[r, pl.ds(c0, 16)] * rv

        pltpu.sync_copy(accw, out_hbm.at[pl.ds(lo, WROWS)])


def _edge_phase_sc(h, aa, cbound, srcp, dstp):
    aa = jnp.pad(aa, ((0, N_PAD - N_NODES), (0, 0)))
    mesh = plsc.VectorSubcoreMesh(core_axis_name="c", subcore_axis_name="s")
    cp = pltpu.CompilerParams()
    if "needs_layout_passes" in pltpu.CompilerParams.__dataclass_fields__:
        cp = dataclasses.replace(cp, needs_layout_passes=False)
    kfn = pl.kernel(
        _edge_body,
        out_type=jax.ShapeDtypeStruct((N_PAD, HC), jnp.float32),
        mesh=mesh,
        compiler_params=cp,
        scratch_types=[
            pltpu.VMEM((SEG,), jnp.int32),
            pltpu.VMEM((SEG,), jnp.int32),
            pltpu.VMEM((SEG,), jnp.int32),
            pltpu.VMEM((SEG,), jnp.int32),
            pltpu.VMEM((MCAP + 16,), jnp.int32),
            pltpu.VMEM((MCAP + 16,), jnp.int32),
            pltpu.VMEM((16,), jnp.int32),
            pltpu.VMEM((16,), jnp.int32),
            pltpu.VMEM((8, 128), jnp.float32),
            pltpu.VMEM((16, 128), jnp.float32),
            pltpu.VMEM((8, 128), jnp.float32),
            pltpu.VMEM((16, 128), jnp.float32),
            pltpu.VMEM((8, 32), jnp.float32),
            pltpu.VMEM((16,), jnp.float32),
            pltpu.VMEM((8, HC), jnp.float32),
            pltpu.VMEM((8, HC), jnp.float32),
            pltpu.VMEM((WROWS, HC), jnp.float32),
            pltpu.VMEM((WROWS, 16), jnp.float32),
            pltpu.SemaphoreType.DMA,
            pltpu.SemaphoreType.DMA,
            pltpu.SemaphoreType.DMA,
            pltpu.SemaphoreType.DMA,
        ],
    )
    return kfn(h, aa, cbound, srcp, dstp)[0:N_NODES]


def _build_acat(att_src, att_dst):
    eye = jnp.eye(HEADS, dtype=jnp.float32)
    a_src = (att_src[:, None, :] * eye[:, :, None]).transpose(0, 2, 1).reshape(HC, HEADS)
    a_dst = (att_dst[:, None, :] * eye[:, :, None]).transpose(0, 2, 1).reshape(HC, HEADS)
    return jnp.concatenate([jnp.zeros((HC, 1), jnp.float32), a_src, a_dst,
                            jnp.zeros((HC, 3), jnp.float32)], axis=1)


def _cbound(cm):
    c6 = cm[0, 1:7] + cm[0, 7:13]
    c6 = jnp.where(c6 > 0, c6, 0.2 * c6)
    return jnp.pad(c6, (8, 2))


def kernel(x, edge_index, W1, att_src1, att_dst1, b1, W2, att_src2, att_dst2, b2):
    n = x.shape[0]
    loop = jnp.arange(n, dtype=jnp.int32)
    pad = E_PAD - E_TOT
    srcp = jnp.concatenate([edge_index[0].astype(jnp.int32), loop,
                            jnp.zeros((pad,), jnp.int32)])
    dstp = jnp.concatenate([edge_index[1].astype(jnp.int32), loop,
                            jnp.full((pad,), 2 * N_NODES, jnp.int32)])

    A1 = _build_acat(att_src1, att_dst1)
    A2 = _build_acat(att_src2, att_dst2)

    h1, aa1, cm1 = _linear_stage(x, W1, A1)
    acc1 = _edge_phase_sc(h1, aa1, _cbound(cm1), srcp, dstp)
    h2, aa2, cm2 = _mid_stage(acc1, b1.reshape(1, CH), W2, A2)
    acc2 = _edge_phase_sc(h2, aa2, _cbound(cm2), srcp, dstp)
    return _final_stage(acc2, b2.reshape(1, CH))
